# baseline (speedup 1.0000x reference)
import functools

import jax
import jax.numpy as jnp
from jax import lax
from jax.experimental import pallas as pl
from jax.experimental.pallas import tpu as pltpu
from jax.experimental.pallas import tpu_sc as plsc

DEPTH = 24
VQT_NUM = 10
EMBED_DIM = 1024
BATCH = 256
ROW = VQT_NUM * EMBED_DIM

_info = plsc.get_sparse_core_info()
_NC = _info.num_cores
_NS = _info.num_subcores
_NW = _NC * _NS
_B_PER_W = BATCH // _NW

_mesh = plsc.VectorSubcoreMesh(core_axis_name="c", subcore_axis_name="s")


@functools.partial(
    pl.kernel,
    mesh=_mesh,
    out_type=jax.ShapeDtypeStruct((BATCH, ROW), jnp.float32),
    scratch_types=[
        pltpu.VMEM((1,), jnp.int32),
        pltpu.VMEM((1, ROW), jnp.float32),
        pltpu.SemaphoreType.DMA,
        pltpu.SemaphoreType.DMA,
    ],
)
def _vqt_expand(table_hbm, idx_hbm, out_hbm, idx_v, row_v, gsem, wsem):
    wid = lax.axis_index("s") * _NC + lax.axis_index("c")
    base = wid * _B_PER_W
    pltpu.sync_copy(idx_hbm, idx_v)
    pltpu.async_copy(table_hbm.at[idx_v], row_v, gsem).wait()
    copies = [
        pltpu.async_copy(row_v, out_hbm.at[pl.ds(base + j, 1)], wsem)
        for j in range(_B_PER_W)
    ]
    for c in copies:
        c.wait()


def kernel(query_prompt_embeddings, index, batch_size):
    del batch_size
    table = query_prompt_embeddings.reshape(DEPTH, ROW)
    idx = jnp.asarray(index, jnp.int32).reshape(1)
    out = _vqt_expand(table, idx)
    return out.reshape(BATCH, VQT_NUM, EMBED_DIM)


# speedup vs baseline: 1.0144x; 1.0144x over previous
import functools

import jax
import jax.numpy as jnp
from jax import lax
from jax.experimental import pallas as pl
from jax.experimental.pallas import tpu as pltpu
from jax.experimental.pallas import tpu_sc as plsc

DEPTH = 24
VQT_NUM = 10
EMBED_DIM = 1024
BATCH = 256

_info = plsc.get_sparse_core_info()
_NC = _info.num_cores
_NS = _info.num_subcores
_NL = _info.num_lanes
_NW = _NC * _NS
_B_PER_W = BATCH // _NW

_mesh = plsc.VectorSubcoreMesh(core_axis_name="c", subcore_axis_name="s")


@functools.partial(
    pl.kernel,
    mesh=_mesh,
    out_type=jax.ShapeDtypeStruct((BATCH, VQT_NUM, EMBED_DIM), jnp.float32),
    scratch_types=[
        pltpu.VMEM((_NL,), jnp.int32),
        pltpu.VMEM((VQT_NUM, EMBED_DIM), jnp.float32),
        pltpu.SemaphoreType.DMA,
        pltpu.SemaphoreType.DMA,
    ],
)
def _vqt_expand(table_hbm, idx_hbm, out_hbm, idx_v, row_v, gsem, wsem):
    wid = lax.axis_index("s") * _NC + lax.axis_index("c")
    base = wid * _B_PER_W
    pltpu.sync_copy(idx_hbm, idx_v)
    layer = idx_v[...][0]
    pltpu.async_copy(table_hbm.at[layer], row_v, gsem).wait()
    copies = [
        pltpu.async_copy(row_v, out_hbm.at[base + j], wsem)
        for j in range(_B_PER_W)
    ]
    for c in copies:
        c.wait()


def kernel(query_prompt_embeddings, index, batch_size):
    del batch_size
    idx = jnp.zeros((_NL,), jnp.int32).at[0].set(index)
    return _vqt_expand(query_prompt_embeddings, idx)


# speedup vs baseline: 1.5739x; 1.5515x over previous
import functools

import jax
import jax.numpy as jnp
from jax import lax
from jax.experimental import pallas as pl
from jax.experimental.pallas import tpu as pltpu
from jax.experimental.pallas import tpu_sc as plsc

DEPTH = 24
VQT_NUM = 10
EMBED_DIM = 1024
BATCH = 256

_info = plsc.get_sparse_core_info()
_NC = _info.num_cores
_NS = _info.num_subcores
_NL = _info.num_lanes
_NW = _NC * _NS
_B_PER_W = BATCH // _NW

_mesh = plsc.VectorSubcoreMesh(core_axis_name="c", subcore_axis_name="s")


@functools.partial(
    pl.kernel,
    mesh=_mesh,
    out_type=jax.ShapeDtypeStruct((VQT_NUM, BATCH, EMBED_DIM), jnp.float32),
    scratch_types=[
        pltpu.VMEM((_NL,), jnp.int32),
        pltpu.VMEM((VQT_NUM, EMBED_DIM), jnp.float32),
        pltpu.SemaphoreType.DMA,
        pltpu.SemaphoreType.DMA,
    ],
)
def _vqt_expand(table_hbm, idx_hbm, out_hbm, idx_v, row_v, gsem, wsem):
    wid = lax.axis_index("s") * _NC + lax.axis_index("c")
    base = wid * _B_PER_W
    pltpu.sync_copy(idx_hbm, idx_v)
    layer = idx_v[...][0]
    pltpu.async_copy(table_hbm.at[layer], row_v, gsem).wait()
    copies = [
        pltpu.async_copy(row_v.at[v], out_hbm.at[v].at[base + j], wsem)
        for v in range(VQT_NUM)
        for j in range(_B_PER_W)
    ]
    for c in copies:
        c.wait()


def kernel(query_prompt_embeddings, index, batch_size):
    del batch_size
    idx = jnp.zeros((_NL,), jnp.int32).at[0].set(index)
    out = _vqt_expand(query_prompt_embeddings, idx)
    return jnp.transpose(out, (1, 0, 2))
